# baseline (speedup 1.0000x reference)
import jax
import jax.numpy as jnp
from jax.experimental import pallas as pl

_BM = 256
_STREAMS = 4


def _router_block(*refs):
    w_ref = refs[_STREAMS]
    o_ref = refs[_STREAMS + 1]
    for s in range(_STREAMS):
        o_ref[s * _BM:(s + 1) * _BM, :] = jax.lax.dot_general(
            refs[s][...], w_ref[...],
            dimension_numbers=(((1,), (1,)), ((), ())),
            preferred_element_type=jnp.float32,
            precision=jax.lax.Precision.DEFAULT,
        )


def kernel(hidden_states, W):
    hs = hidden_states.reshape(-1, hidden_states.shape[-1])
    m, k = hs.shape
    n = W.shape[0]
    bo = _BM * _STREAMS

    def stripe(s):
        return pl.BlockSpec((_BM, k), lambda i, s=s: (i * _STREAMS + s, 0))

    return pl.pallas_call(
        _router_block,
        grid=(m // bo,),
        in_specs=[stripe(s) for s in range(_STREAMS)]
        + [pl.BlockSpec((n, k), lambda i: (0, 0))],
        out_specs=pl.BlockSpec((bo, n), lambda i: (i, 0)),
        out_shape=jax.ShapeDtypeStruct((m, n), jnp.float32),
    )(*([hs] * _STREAMS), W)
